# Initial kernel scaffold; baseline (speedup 1.0000x reference)
import jax
import jax.numpy as jnp
from jax.experimental import pallas as pl


def kernel(predicted_velocities, target_velocities, positions, obstacles, boundary_mask):
    raise NotImplementedError("write your pallas kernel here")


# speedup vs baseline: 16.1494x; 16.1494x over previous
import functools

import jax
import jax.numpy as jnp
import numpy as np
from jax.experimental import pallas as pl

_VEL_W, _CON_W, _BND_W, _OBS_W, _DIV_W = 1.0, 0.1, 0.5, 1.0, 0.1
_B, _N, _M = 4, 2048, 16
_K = 5
_R = 128
_NB = _N // _R
_S = 100
_OBW = 128


def _div_onehots():
    rng = np.random.default_rng(0)
    idx = np.stack([rng.permutation(_N)[:4] for _ in range(_S)])
    o = np.zeros((3, _N, 128), np.float32)
    for s in range(_S):
        for j in range(3):
            o[j, idx[s, j], s] = 1.0
    return o


_ONEHOTS = _div_onehots()


def _loss_body(rows_ref, cols_ref, rowsn_ref, obs_ref, o0_ref, o1_ref,
               o2_ref, out_ref):
    i = pl.program_id(1)
    rows = rows_ref[0]
    px_i = rows[:, 0:1]
    py_i = rows[:, 1:2]
    vx_i = rows[:, 2:3]
    vy_i = rows[:, 3:4]
    tx_i = rows[:, 4:5]
    ty_i = rows[:, 5:6]
    msk = rows[:, 6:7]

    cols = cols_ref[0]
    px_j = cols[0:1, :]
    py_j = cols[1:2, :]
    vx_j = cols[2:3, :]
    vy_j = cols[3:4, :]

    dx = px_i - px_j
    dy = py_i - py_j
    d = jnp.sqrt(dx * dx + dy * dy + 1e-12)
    wx = vx_i - vx_j
    wy = vy_i - vy_j
    vsq = wx * wx + wy * wy

    col_ids = jax.lax.broadcasted_iota(jnp.int32, (1, _N), 1)
    row_ids = i * _R + jax.lax.broadcasted_iota(jnp.int32, (_R, 1), 0)
    big = jnp.float32(1e6)
    dns = jnp.where(col_ids == row_ids, big, d)

    acc = jnp.zeros((_R, 1), jnp.float32)
    for _ in range(_K):
        dmin = jnp.min(dns, axis=1, keepdims=True)
        eq = dns == dmin
        jmin = jnp.min(jnp.where(eq, col_ids, jnp.int32(_N)),
                       axis=1, keepdims=True)
        sel = col_ids == jmin
        vsel = jnp.sum(jnp.where(sel, vsq, 0.0), axis=1, keepdims=True)
        vd = jnp.sqrt(vsel + 1e-12)
        acc = acc + vd * (1.0 / (dmin + 1e-6))
        dns = jnp.where(sel, big, dns)
    con_part = jnp.sum(acc)

    vl_part = jnp.sum((vx_i - tx_i) ** 2 + (vy_i - ty_i) ** 2)

    a0, a1, a2, a3 = px_i, 1.0 - px_i, py_i, 1.0 - py_i
    is0 = (a0 <= a1) & (a0 <= a2) & (a0 <= a3)
    is1 = (~is0) & (a1 <= a2) & (a1 <= a3)
    is2 = (~is0) & (~is1) & (a2 <= a3)
    is3 = (~is0) & (~is1) & (~is2)
    nx = jnp.where(is0, -1.0, jnp.where(is1, 1.0, 0.0))
    ny = jnp.where(is2, -1.0, jnp.where(is3, 1.0, 0.0))
    nc = vx_i * nx + vy_i * ny
    bl_num = jnp.sum(nc * nc * msk)
    bl_cnt = jnp.sum(msk)

    cx = obs_ref[0, 0:1, :]
    cy = obs_ref[0, 1:2, :]
    rr = obs_ref[0, 2:3, :]
    dxo = px_i - cx
    dyo = py_i - cy
    disto = jnp.sqrt(dxo * dxo + dyo * dyo + 1e-12)
    near = (disto < rr * 2.0).astype(jnp.float32)
    wexp = jnp.exp(-(disto - rr) / (rr * 0.5))
    proj = (vx_i * dxo + vy_i * dyo) / (disto + 1e-6)
    pen = wexp * jnp.maximum(-proj, 0.0) ** 2
    pns = jnp.sum(pen * near, axis=0, keepdims=True)
    ncnt = jnp.sum(near, axis=0, keepdims=True)

    def bc(s):
        return jnp.broadcast_to(jnp.reshape(s, (1, 1)), (1, 128))

    tile = jnp.concatenate(
        [bc(con_part), bc(vl_part), bc(bl_num), bc(bl_cnt),
         jnp.zeros((1, 128), jnp.float32),
         pns, ncnt, jnp.zeros((1, 128), jnp.float32)], axis=0)
    out_ref[0, 0] = tile

    @pl.when(i == 0)
    def _divergence():
        rn = rowsn_ref[0]
        o0 = o0_ref[...]
        o1 = o1_ref[...]
        o2 = o2_ref[...]

        def pick(c, o):
            return jnp.sum(rn[:, c:c + 1] * o, axis=0, keepdims=True)

        p0x, p0y = pick(0, o0), pick(1, o0)
        v0x, v0y = pick(2, o0), pick(3, o0)
        p1x, v1x = pick(0, o1), pick(2, o1)
        p2y, v2y = pick(1, o2), pick(3, o2)
        dxs = p1x - p0x
        dys = p2y - p0y
        dvx = v1x - v0x
        dvy = v2y - v0y
        div = dvx / (dxs + 1e-6) + dvy / (dys + 1e-6)
        out_ref[0, 0, 4:5, :] = bc(jnp.sum(div * div))


@jax.jit
def kernel(predicted_velocities, target_velocities, positions, obstacles,
           boundary_mask):
    mask_f = boundary_mask.astype(jnp.float32)[..., None]
    zeros_rows = jnp.zeros((_B, _N, 1), jnp.float32)
    rows = jnp.concatenate(
        [positions, predicted_velocities, target_velocities, mask_f,
         zeros_rows], axis=-1)
    cols = jnp.concatenate(
        [jnp.transpose(positions, (0, 2, 1)),
         jnp.transpose(predicted_velocities, (0, 2, 1)),
         jnp.zeros((_B, 4, _N), jnp.float32)], axis=1)
    obs_p = jnp.zeros((_B, 8, _OBW), jnp.float32)
    obs_p = obs_p.at[:, 0:3, 0:_M].set(jnp.transpose(obstacles, (0, 2, 1)))

    o0 = jnp.asarray(_ONEHOTS[0])
    o1 = jnp.asarray(_ONEHOTS[1])
    o2 = jnp.asarray(_ONEHOTS[2])

    parts = pl.pallas_call(
        _loss_body,
        grid=(_B, _NB),
        in_specs=[
            pl.BlockSpec((1, _R, 8), lambda b, i: (b, i, 0)),
            pl.BlockSpec((1, 8, _N), lambda b, i: (b, 0, 0)),
            pl.BlockSpec((1, _N, 8), lambda b, i: (b, 0, 0)),
            pl.BlockSpec((1, 8, _OBW), lambda b, i: (b, 0, 0)),
            pl.BlockSpec((_N, 128), lambda b, i: (0, 0)),
            pl.BlockSpec((_N, 128), lambda b, i: (0, 0)),
            pl.BlockSpec((_N, 128), lambda b, i: (0, 0)),
        ],
        out_specs=pl.BlockSpec((1, 1, 8, 128), lambda b, i: (b, i, 0, 0)),
        out_shape=jax.ShapeDtypeStruct((_B, _NB, 8, 128), jnp.float32),
    )(rows, cols, rows, obs_p, o0, o1, o2)

    cl = parts[:, :, 0, 0].sum() / (_B * _N * _K)
    vl = parts[:, :, 1, 0].sum() / (_B * _N * 2)
    bln = parts[:, :, 2, 0].sum()
    blc = parts[:, :, 3, 0].sum()
    bl = jnp.where(blc > 0, bln / jnp.maximum(blc, 1.0), 0.0)
    dl = parts[:, :, 4, 0].sum() / (_B * _S)
    pns = parts[:, :, 5, 0:_M].sum(axis=1)
    cnt = parts[:, :, 6, 0:_M].sum(axis=1)
    rr = obstacles[:, :, 2]
    penalty = jnp.where(cnt > 0, pns / jnp.maximum(cnt, 1.0), 0.0)
    penalty = penalty * (rr > 0).astype(jnp.float32)
    ol = penalty.sum() / (_B * _M)

    return (_VEL_W * vl + _CON_W * cl + _BND_W * bl + _OBS_W * ol
            + _DIV_W * dl)


# speedup vs baseline: 17.3816x; 1.0763x over previous
import functools

import jax
import jax.numpy as jnp
import numpy as np
from jax.experimental import pallas as pl

_VEL_W, _CON_W, _BND_W, _OBS_W, _DIV_W = 1.0, 0.1, 0.5, 1.0, 0.1
_B, _N, _M = 4, 2048, 16
_K = 5
_R = 256
_NB = _N // _R
_S = 100
_OBW = 128


def _div_onehots():
    rng = np.random.default_rng(0)
    idx = np.stack([rng.permutation(_N)[:4] for _ in range(_S)])
    o = np.zeros((3, _N, 128), np.float32)
    for s in range(_S):
        for j in range(3):
            o[j, idx[s, j], s] = 1.0
    return o


_ONEHOTS = _div_onehots()


def _loss_body(rows_ref, cols_ref, rowsn_ref, obs_ref, o0_ref, o1_ref,
               o2_ref, out_ref):
    i = pl.program_id(1)
    rows = rows_ref[0]
    px_i = rows[:, 0:1]
    py_i = rows[:, 1:2]
    vx_i = rows[:, 2:3]
    vy_i = rows[:, 3:4]
    tx_i = rows[:, 4:5]
    ty_i = rows[:, 5:6]
    msk = rows[:, 6:7]

    cols = cols_ref[0]
    px_j = cols[0:1, :]
    py_j = cols[1:2, :]
    vx_j = cols[2:3, :]
    vy_j = cols[3:4, :]

    dx = px_i - px_j
    dy = py_i - py_j
    d = jnp.sqrt(dx * dx + dy * dy + 1e-12)
    wx = vx_i - vx_j
    wy = vy_i - vy_j
    vsq = wx * wx + wy * wy

    col_ids = jax.lax.broadcasted_iota(jnp.int32, (1, _N), 1)
    row_ids = i * _R + jax.lax.broadcasted_iota(jnp.int32, (_R, 1), 0)
    big = jnp.float32(1e6)
    dns = jnp.where(col_ids == row_ids, big, d)

    acc = jnp.zeros((_R, 1), jnp.float32)
    for _ in range(_K):
        dmin = jnp.min(dns, axis=1, keepdims=True)
        eq = dns == dmin
        jmin = jnp.min(jnp.where(eq, col_ids, jnp.int32(_N)),
                       axis=1, keepdims=True)
        sel = col_ids == jmin
        vsel = jnp.sum(jnp.where(sel, vsq, 0.0), axis=1, keepdims=True)
        vd = jnp.sqrt(vsel + 1e-12)
        acc = acc + vd * (1.0 / (dmin + 1e-6))
        dns = jnp.where(sel, big, dns)
    con_part = jnp.sum(acc)

    vl_part = jnp.sum((vx_i - tx_i) ** 2 + (vy_i - ty_i) ** 2)

    a0, a1, a2, a3 = px_i, 1.0 - px_i, py_i, 1.0 - py_i
    is0 = (a0 <= a1) & (a0 <= a2) & (a0 <= a3)
    is1 = (~is0) & (a1 <= a2) & (a1 <= a3)
    is2 = (~is0) & (~is1) & (a2 <= a3)
    is3 = (~is0) & (~is1) & (~is2)
    nx = jnp.where(is0, -1.0, jnp.where(is1, 1.0, 0.0))
    ny = jnp.where(is2, -1.0, jnp.where(is3, 1.0, 0.0))
    nc = vx_i * nx + vy_i * ny
    bl_num = jnp.sum(nc * nc * msk)
    bl_cnt = jnp.sum(msk)

    cx = obs_ref[0, 0:1, :]
    cy = obs_ref[0, 1:2, :]
    rr = obs_ref[0, 2:3, :]
    dxo = px_i - cx
    dyo = py_i - cy
    disto = jnp.sqrt(dxo * dxo + dyo * dyo + 1e-12)
    near = (disto < rr * 2.0).astype(jnp.float32)
    wexp = jnp.exp(-(disto - rr) / (rr * 0.5))
    proj = (vx_i * dxo + vy_i * dyo) / (disto + 1e-6)
    pen = wexp * jnp.maximum(-proj, 0.0) ** 2
    pns = jnp.sum(pen * near, axis=0, keepdims=True)
    ncnt = jnp.sum(near, axis=0, keepdims=True)

    def bc(s):
        return jnp.broadcast_to(jnp.reshape(s, (1, 1)), (1, 128))

    tile = jnp.concatenate(
        [bc(con_part), bc(vl_part), bc(bl_num), bc(bl_cnt),
         jnp.zeros((1, 128), jnp.float32),
         pns, ncnt, jnp.zeros((1, 128), jnp.float32)], axis=0)
    out_ref[0, 0] = tile

    @pl.when(i == 0)
    def _divergence():
        rn = rowsn_ref[0]
        o0 = o0_ref[...]
        o1 = o1_ref[...]
        o2 = o2_ref[...]

        def pick(c, o):
            return jnp.sum(rn[:, c:c + 1] * o, axis=0, keepdims=True)

        p0x, p0y = pick(0, o0), pick(1, o0)
        v0x, v0y = pick(2, o0), pick(3, o0)
        p1x, v1x = pick(0, o1), pick(2, o1)
        p2y, v2y = pick(1, o2), pick(3, o2)
        dxs = p1x - p0x
        dys = p2y - p0y
        dvx = v1x - v0x
        dvy = v2y - v0y
        div = dvx / (dxs + 1e-6) + dvy / (dys + 1e-6)
        out_ref[0, 0, 4:5, :] = bc(jnp.sum(div * div))


@jax.jit
def kernel(predicted_velocities, target_velocities, positions, obstacles,
           boundary_mask):
    mask_f = boundary_mask.astype(jnp.float32)[..., None]
    zeros_rows = jnp.zeros((_B, _N, 1), jnp.float32)
    rows = jnp.concatenate(
        [positions, predicted_velocities, target_velocities, mask_f,
         zeros_rows], axis=-1)
    cols = jnp.concatenate(
        [jnp.transpose(positions, (0, 2, 1)),
         jnp.transpose(predicted_velocities, (0, 2, 1)),
         jnp.zeros((_B, 4, _N), jnp.float32)], axis=1)
    obs_p = jnp.zeros((_B, 8, _OBW), jnp.float32)
    obs_p = obs_p.at[:, 0:3, 0:_M].set(jnp.transpose(obstacles, (0, 2, 1)))

    o0 = jnp.asarray(_ONEHOTS[0])
    o1 = jnp.asarray(_ONEHOTS[1])
    o2 = jnp.asarray(_ONEHOTS[2])

    parts = pl.pallas_call(
        _loss_body,
        grid=(_B, _NB),
        in_specs=[
            pl.BlockSpec((1, _R, 8), lambda b, i: (b, i, 0)),
            pl.BlockSpec((1, 8, _N), lambda b, i: (b, 0, 0)),
            pl.BlockSpec((1, _N, 8), lambda b, i: (b, 0, 0)),
            pl.BlockSpec((1, 8, _OBW), lambda b, i: (b, 0, 0)),
            pl.BlockSpec((_N, 128), lambda b, i: (0, 0)),
            pl.BlockSpec((_N, 128), lambda b, i: (0, 0)),
            pl.BlockSpec((_N, 128), lambda b, i: (0, 0)),
        ],
        out_specs=pl.BlockSpec((1, 1, 8, 128), lambda b, i: (b, i, 0, 0)),
        out_shape=jax.ShapeDtypeStruct((_B, _NB, 8, 128), jnp.float32),
    )(rows, cols, rows, obs_p, o0, o1, o2)

    cl = parts[:, :, 0, 0].sum() / (_B * _N * _K)
    vl = parts[:, :, 1, 0].sum() / (_B * _N * 2)
    bln = parts[:, :, 2, 0].sum()
    blc = parts[:, :, 3, 0].sum()
    bl = jnp.where(blc > 0, bln / jnp.maximum(blc, 1.0), 0.0)
    dl = parts[:, :, 4, 0].sum() / (_B * _S)
    pns = parts[:, :, 5, 0:_M].sum(axis=1)
    cnt = parts[:, :, 6, 0:_M].sum(axis=1)
    rr = obstacles[:, :, 2]
    penalty = jnp.where(cnt > 0, pns / jnp.maximum(cnt, 1.0), 0.0)
    penalty = penalty * (rr > 0).astype(jnp.float32)
    ol = penalty.sum() / (_B * _M)

    return (_VEL_W * vl + _CON_W * cl + _BND_W * bl + _OBS_W * ol
            + _DIV_W * dl)


# speedup vs baseline: 17.6179x; 1.0136x over previous
import functools

import jax
import jax.numpy as jnp
import numpy as np
from jax.experimental import pallas as pl

_VEL_W, _CON_W, _BND_W, _OBS_W, _DIV_W = 1.0, 0.1, 0.5, 1.0, 0.1
_B, _N, _M = 4, 2048, 16
_K = 5
_R = 512
_NB = _N // _R
_S = 100
_OBW = 128


def _div_onehots():
    rng = np.random.default_rng(0)
    idx = np.stack([rng.permutation(_N)[:4] for _ in range(_S)])
    o = np.zeros((3, _N, 128), np.float32)
    for s in range(_S):
        for j in range(3):
            o[j, idx[s, j], s] = 1.0
    return o


_ONEHOTS = _div_onehots()


def _loss_body(rows_ref, cols_ref, rowsn_ref, obs_ref, o0_ref, o1_ref,
               o2_ref, out_ref):
    i = pl.program_id(1)
    rows = rows_ref[0]
    px_i = rows[:, 0:1]
    py_i = rows[:, 1:2]
    vx_i = rows[:, 2:3]
    vy_i = rows[:, 3:4]
    tx_i = rows[:, 4:5]
    ty_i = rows[:, 5:6]
    msk = rows[:, 6:7]

    cols = cols_ref[0]
    px_j = cols[0:1, :]
    py_j = cols[1:2, :]
    vx_j = cols[2:3, :]
    vy_j = cols[3:4, :]

    dx = px_i - px_j
    dy = py_i - py_j
    d = jnp.sqrt(dx * dx + dy * dy + 1e-12)
    wx = vx_i - vx_j
    wy = vy_i - vy_j
    vsq = wx * wx + wy * wy

    col_ids = jax.lax.broadcasted_iota(jnp.int32, (1, _N), 1)
    row_ids = i * _R + jax.lax.broadcasted_iota(jnp.int32, (_R, 1), 0)
    big = jnp.float32(1e6)
    dns = jnp.where(col_ids == row_ids, big, d)

    acc = jnp.zeros((_R, 1), jnp.float32)
    for _ in range(_K):
        dmin = jnp.min(dns, axis=1, keepdims=True)
        eq = dns == dmin
        jmin = jnp.min(jnp.where(eq, col_ids, jnp.int32(_N)),
                       axis=1, keepdims=True)
        sel = col_ids == jmin
        vsel = jnp.sum(jnp.where(sel, vsq, 0.0), axis=1, keepdims=True)
        vd = jnp.sqrt(vsel + 1e-12)
        acc = acc + vd * (1.0 / (dmin + 1e-6))
        dns = jnp.where(sel, big, dns)
    con_part = jnp.sum(acc)

    vl_part = jnp.sum((vx_i - tx_i) ** 2 + (vy_i - ty_i) ** 2)

    a0, a1, a2, a3 = px_i, 1.0 - px_i, py_i, 1.0 - py_i
    is0 = (a0 <= a1) & (a0 <= a2) & (a0 <= a3)
    is1 = (~is0) & (a1 <= a2) & (a1 <= a3)
    is2 = (~is0) & (~is1) & (a2 <= a3)
    is3 = (~is0) & (~is1) & (~is2)
    nx = jnp.where(is0, -1.0, jnp.where(is1, 1.0, 0.0))
    ny = jnp.where(is2, -1.0, jnp.where(is3, 1.0, 0.0))
    nc = vx_i * nx + vy_i * ny
    bl_num = jnp.sum(nc * nc * msk)
    bl_cnt = jnp.sum(msk)

    cx = obs_ref[0, 0:1, :]
    cy = obs_ref[0, 1:2, :]
    rr = obs_ref[0, 2:3, :]
    dxo = px_i - cx
    dyo = py_i - cy
    disto = jnp.sqrt(dxo * dxo + dyo * dyo + 1e-12)
    near = (disto < rr * 2.0).astype(jnp.float32)
    wexp = jnp.exp(-(disto - rr) / (rr * 0.5))
    proj = (vx_i * dxo + vy_i * dyo) / (disto + 1e-6)
    pen = wexp * jnp.maximum(-proj, 0.0) ** 2
    pns = jnp.sum(pen * near, axis=0, keepdims=True)
    ncnt = jnp.sum(near, axis=0, keepdims=True)

    def bc(s):
        return jnp.broadcast_to(jnp.reshape(s, (1, 1)), (1, 128))

    tile = jnp.concatenate(
        [bc(con_part), bc(vl_part), bc(bl_num), bc(bl_cnt),
         jnp.zeros((1, 128), jnp.float32),
         pns, ncnt, jnp.zeros((1, 128), jnp.float32)], axis=0)
    out_ref[0, 0] = tile

    @pl.when(i == 0)
    def _divergence():
        rn = rowsn_ref[0]
        o0 = o0_ref[...]
        o1 = o1_ref[...]
        o2 = o2_ref[...]

        def pick(c, o):
            return jnp.sum(rn[:, c:c + 1] * o, axis=0, keepdims=True)

        p0x, p0y = pick(0, o0), pick(1, o0)
        v0x, v0y = pick(2, o0), pick(3, o0)
        p1x, v1x = pick(0, o1), pick(2, o1)
        p2y, v2y = pick(1, o2), pick(3, o2)
        dxs = p1x - p0x
        dys = p2y - p0y
        dvx = v1x - v0x
        dvy = v2y - v0y
        div = dvx / (dxs + 1e-6) + dvy / (dys + 1e-6)
        out_ref[0, 0, 4:5, :] = bc(jnp.sum(div * div))


@jax.jit
def kernel(predicted_velocities, target_velocities, positions, obstacles,
           boundary_mask):
    mask_f = boundary_mask.astype(jnp.float32)[..., None]
    zeros_rows = jnp.zeros((_B, _N, 1), jnp.float32)
    rows = jnp.concatenate(
        [positions, predicted_velocities, target_velocities, mask_f,
         zeros_rows], axis=-1)
    cols = jnp.concatenate(
        [jnp.transpose(positions, (0, 2, 1)),
         jnp.transpose(predicted_velocities, (0, 2, 1)),
         jnp.zeros((_B, 4, _N), jnp.float32)], axis=1)
    obs_p = jnp.zeros((_B, 8, _OBW), jnp.float32)
    obs_p = obs_p.at[:, 0:3, 0:_M].set(jnp.transpose(obstacles, (0, 2, 1)))

    o0 = jnp.asarray(_ONEHOTS[0])
    o1 = jnp.asarray(_ONEHOTS[1])
    o2 = jnp.asarray(_ONEHOTS[2])

    parts = pl.pallas_call(
        _loss_body,
        grid=(_B, _NB),
        in_specs=[
            pl.BlockSpec((1, _R, 8), lambda b, i: (b, i, 0)),
            pl.BlockSpec((1, 8, _N), lambda b, i: (b, 0, 0)),
            pl.BlockSpec((1, _N, 8), lambda b, i: (b, 0, 0)),
            pl.BlockSpec((1, 8, _OBW), lambda b, i: (b, 0, 0)),
            pl.BlockSpec((_N, 128), lambda b, i: (0, 0)),
            pl.BlockSpec((_N, 128), lambda b, i: (0, 0)),
            pl.BlockSpec((_N, 128), lambda b, i: (0, 0)),
        ],
        out_specs=pl.BlockSpec((1, 1, 8, 128), lambda b, i: (b, i, 0, 0)),
        out_shape=jax.ShapeDtypeStruct((_B, _NB, 8, 128), jnp.float32),
    )(rows, cols, rows, obs_p, o0, o1, o2)

    cl = parts[:, :, 0, 0].sum() / (_B * _N * _K)
    vl = parts[:, :, 1, 0].sum() / (_B * _N * 2)
    bln = parts[:, :, 2, 0].sum()
    blc = parts[:, :, 3, 0].sum()
    bl = jnp.where(blc > 0, bln / jnp.maximum(blc, 1.0), 0.0)
    dl = parts[:, :, 4, 0].sum() / (_B * _S)
    pns = parts[:, :, 5, 0:_M].sum(axis=1)
    cnt = parts[:, :, 6, 0:_M].sum(axis=1)
    rr = obstacles[:, :, 2]
    penalty = jnp.where(cnt > 0, pns / jnp.maximum(cnt, 1.0), 0.0)
    penalty = penalty * (rr > 0).astype(jnp.float32)
    ol = penalty.sum() / (_B * _M)

    return (_VEL_W * vl + _CON_W * cl + _BND_W * bl + _OBS_W * ol
            + _DIV_W * dl)
